# Initial kernel scaffold; baseline (speedup 1.0000x reference)
import jax
import jax.numpy as jnp
from jax.experimental import pallas as pl


def kernel(x_nchw, w1, b1, w2, b2, w3, b3, w4, b4, w5, b5, wfc1, bfc1, wfc2, bfc2, wout, bout):
    raise NotImplementedError("write your pallas kernel here")


# speedup vs baseline: 2.4015x; 2.4015x over previous
import jax
import jax.numpy as jnp
from jax.experimental import pallas as pl
from jax.experimental.pallas import tpu as pltpu

_BF = jnp.bfloat16
_F32 = jnp.float32


def _pool2(a):
    s = a.shape
    a = a.reshape(*s[:-3], s[-3] // 2, 2, s[-2] // 2, 2, s[-1])
    return jnp.max(jnp.max(a, axis=-2), axis=-3)


def _pack_w1(w1):
    w_all = jnp.zeros((108, 128), dtype=w1.dtype)
    for u in range(2):
        for vo in range(2):
            for kh in range(5):
                for kw in range(5):
                    di, r = divmod(u + kh, 2)
                    dj, v = divmod(vo + kw, 2)
                    row = di * 36 + dj * 12 + r * 6 + v * 3
                    col = u * 64 + vo * 32
                    w_all = jax.lax.dynamic_update_slice(
                        w_all, w1[kh, kw], (row, col))
    return w_all


def _stage12_kernel(x_ref, w1_ref, b1_ref, w2_ref, b2_ref, o_ref):
    x = x_ref[0].reshape(78, 76, 36)
    patches = jnp.concatenate([x[di:di + 76] for di in range(3)],
                              axis=-1)
    p = jnp.dot(patches.reshape(76 * 76, 108), w1_ref[...],
                preferred_element_type=_F32).reshape(76, 76, 128)
    h1 = jnp.maximum(jnp.maximum(p[:, :, 0:32], p[:, :, 32:64]),
                     jnp.maximum(p[:, :, 64:96], p[:, :, 96:128]))
    h1 = jnp.maximum(h1 + b1_ref[...], 0.0).astype(_BF)
    hp = jnp.pad(h1, ((2, 2), (2, 2), (0, 0)))
    xf = jnp.concatenate([hp[:, kw:kw + 76, :] for kw in range(5)],
                         axis=-1)
    q = jnp.dot(xf.reshape(80 * 76, 160), w2_ref[...],
                preferred_element_type=_F32)
    q = q.reshape(80, 76, 320)
    a2 = (q[0:76, :, 0:64] + q[1:77, :, 64:128] + q[2:78, :, 128:192]
          + q[3:79, :, 192:256] + q[4:80, :, 256:320])
    h2 = _pool2(a2)
    h2 = jnp.maximum(h2 + b2_ref[...], 0.0).astype(_BF)
    o_ref[0] = h2.reshape(38, 38 * 64)


def _conv_block(x, wk, b, Ho, B):
    KH = wk.shape[0]
    Cout = wk.shape[2]
    xf = jnp.concatenate([x[:, :, kw:kw + Ho, :] for kw in range(KH)],
                         axis=-1)
    acc = jnp.dot(xf[:, 0:Ho].reshape(B * Ho * Ho, -1), wk[0],
                  preferred_element_type=_F32)
    for kh in range(1, KH):
        acc = acc + jnp.dot(xf[:, kh:kh + Ho].reshape(B * Ho * Ho, -1), wk[kh],
                            preferred_element_type=_F32)
    h = _pool2(acc.reshape(B, Ho, Ho, Cout))
    return jnp.maximum(h + b, 0.0).astype(_BF)


def _stage345_kernel(x_ref, w3_ref, b3_ref, w4_ref, b4_ref, w5_ref, b5_ref,
                     o_ref):
    x = x_ref[...].reshape(8, 38, 38, 64)
    h3 = _conv_block(x, w3_ref[...], b3_ref[...], 36, 8)
    h4 = _conv_block(h3, w4_ref[...], b4_ref[...], 16, 8)
    h5 = _conv_block(h4, w5_ref[...], b5_ref[...], 6, 8)
    o_ref[...] = h5.reshape(8, 9, 428)


def _head_kernel(x_ref, w1_ref, b1_ref, w2_ref, b2_ref, w3_ref, b3_ref, o_ref):
    h = jnp.dot(x_ref[...], w1_ref[...], preferred_element_type=_F32)
    h = jnp.maximum(h + b1_ref[...], 0.0).astype(_BF)
    h = jnp.dot(h, w2_ref[...], preferred_element_type=_F32)
    h = jnp.maximum(h + b2_ref[...], 0.0).astype(_BF)
    h = jnp.dot(h, w3_ref[...], preferred_element_type=_F32)
    o_ref[...] = (h + b3_ref[...]).astype(o_ref.dtype)


def kernel(x_nchw, w1, b1, w2, b2, w3, b3, w4, b4, w5, b5,
           wfc1, bfc1, wfc2, bfc2, wout, bout):
    N = x_nchw.shape[0]
    x = jnp.transpose(x_nchw, (0, 2, 3, 1))
    xp = jnp.pad(x, ((0, 0), (2, 2), (2, 2), (0, 0))).astype(_BF)
    xs2d = (xp.reshape(N, 78, 2, 78, 2, 3).transpose(0, 1, 3, 2, 4, 5)
            .reshape(N, 78, 78, 12))
    xdj = jnp.concatenate([xs2d[:, :, dj:dj + 76, :] for dj in range(3)],
                          axis=-1).reshape(N, 78, 76 * 36)

    w1p = _pack_w1(w1.astype(_BF))
    w2p = jnp.transpose(w2, (1, 2, 0, 3)).reshape(160, 5 * 64).astype(_BF)
    wk3 = w3.reshape(3, 3 * 64, 128).astype(_BF)
    wk4 = w4.reshape(3, 3 * 128, 256).astype(_BF)
    wk5 = w5.reshape(3, 3 * 256, 428).astype(_BF)

    h2 = pl.pallas_call(
        _stage12_kernel,
        out_shape=jax.ShapeDtypeStruct((N, 38, 38 * 64), _BF),
        grid=(N,),
        in_specs=[
            pl.BlockSpec((1, 78, 2736), lambda n: (n, 0, 0)),
            pl.BlockSpec((108, 128), lambda n: (0, 0)),
            pl.BlockSpec((1, 32), lambda n: (0, 0)),
            pl.BlockSpec((160, 320), lambda n: (0, 0)),
            pl.BlockSpec((1, 64), lambda n: (0, 0)),
        ],
        out_specs=pl.BlockSpec((1, 38, 2432), lambda n: (n, 0, 0)),
        compiler_params=pltpu.CompilerParams(
            dimension_semantics=("parallel",),
            vmem_limit_bytes=100 * 1024 * 1024,
        ),
    )(xdj, w1p, b1.reshape(1, 32), w2p, b2.reshape(1, 64))

    B = 8
    flat = pl.pallas_call(
        _stage345_kernel,
        out_shape=jax.ShapeDtypeStruct((N, 9, 428), _BF),
        grid=(N // B,),
        in_specs=[
            pl.BlockSpec((B, 38, 2432), lambda c: (c, 0, 0)),
            pl.BlockSpec((3, 192, 128), lambda c: (0, 0, 0)),
            pl.BlockSpec((1, 128), lambda c: (0, 0)),
            pl.BlockSpec((3, 384, 256), lambda c: (0, 0, 0)),
            pl.BlockSpec((1, 256), lambda c: (0, 0)),
            pl.BlockSpec((3, 768, 428), lambda c: (0, 0, 0)),
            pl.BlockSpec((1, 428), lambda c: (0, 0)),
        ],
        out_specs=pl.BlockSpec((B, 9, 428), lambda c: (c, 0, 0)),
        compiler_params=pltpu.CompilerParams(
            dimension_semantics=("parallel",),
            vmem_limit_bytes=100 * 1024 * 1024,
        ),
    )(h2, wk3, b3.reshape(1, 128), wk4, b4.reshape(1, 256),
      wk5, b5.reshape(1, 428))

    wfc1p = (wfc1.reshape(428, 9, 1024).transpose(1, 0, 2)
             .reshape(3852, 1024).astype(_BF))
    n_out = wout.shape[1]
    out = pl.pallas_call(
        _head_kernel,
        out_shape=jax.ShapeDtypeStruct((N, n_out), _F32),
        grid=(1,),
        in_specs=[
            pl.BlockSpec((N, 3852), lambda i: (0, 0)),
            pl.BlockSpec((3852, 1024), lambda i: (0, 0)),
            pl.BlockSpec((1, 1024), lambda i: (0, 0)),
            pl.BlockSpec((1024, 128), lambda i: (0, 0)),
            pl.BlockSpec((1, 128), lambda i: (0, 0)),
            pl.BlockSpec((128, n_out), lambda i: (0, 0)),
            pl.BlockSpec((1, n_out), lambda i: (0, 0)),
        ],
        out_specs=pl.BlockSpec((N, n_out), lambda i: (0, 0)),
        compiler_params=pltpu.CompilerParams(
            dimension_semantics=("arbitrary",),
            vmem_limit_bytes=100 * 1024 * 1024,
        ),
    )(flat.reshape(N, 3852), wfc1p, bfc1.reshape(1, -1),
      wfc2.astype(_BF), bfc2.reshape(1, -1), wout.astype(_BF),
      bout.reshape(1, -1))
    return out


# speedup vs baseline: 3.5076x; 1.4606x over previous
import numpy as np

import jax
import jax.numpy as jnp
from jax.experimental import pallas as pl
from jax.experimental.pallas import tpu as pltpu

_BF = jnp.bfloat16
_F32 = jnp.float32


def _build_s2d_matrix():
    S = np.zeros((912, 2736), np.float32)
    for J in range(76):
        for dj in range(3):
            for r in range(2):
                for v in range(2):
                    for c in range(3):
                        j = 2 * (J + dj) + v - 2
                        if 0 <= j < 152:
                            S[c * 304 + r * 152 + j,
                              J * 36 + dj * 12 + r * 6 + v * 3 + c] = 1.0
    return S


_S2D_NP = _build_s2d_matrix()


def _pool2(a):
    s = a.shape
    a = a.reshape(*s[:-3], s[-3] // 2, 2, s[-2] // 2, 2, s[-1])
    return jnp.max(jnp.max(a, axis=-2), axis=-3)


def _pack_w1(w1):
    w_all = jnp.zeros((108, 128), dtype=w1.dtype)
    for u in range(2):
        for vo in range(2):
            for kh in range(5):
                for kw in range(5):
                    di, r = divmod(u + kh, 2)
                    dj, v = divmod(vo + kw, 2)
                    row = di * 36 + dj * 12 + r * 6 + v * 3
                    col = u * 64 + vo * 32
                    w_all = jax.lax.dynamic_update_slice(
                        w_all, w1[kh, kw], (row, col))
    return w_all


def _stage12_kernel(x_ref, s_ref, w1_ref, b1_ref, w2_ref, b2_ref, o_ref):
    xc = x_ref[0]
    a = jnp.concatenate([xc[0], xc[1], xc[2]], axis=-1)
    y = jnp.dot(a, s_ref[...],
                preferred_element_type=_F32).astype(_BF)
    x = jnp.pad(y.reshape(76, 76, 36), ((1, 1), (0, 0), (0, 0)))
    patches = jnp.concatenate([x[di:di + 76] for di in range(3)],
                              axis=-1)
    p = jnp.dot(patches.reshape(76 * 76, 108), w1_ref[...],
                preferred_element_type=_F32).reshape(76, 76, 128)
    h1 = jnp.maximum(jnp.maximum(p[:, :, 0:32], p[:, :, 32:64]),
                     jnp.maximum(p[:, :, 64:96], p[:, :, 96:128]))
    h1 = jnp.maximum(h1 + b1_ref[...], 0.0).astype(_BF)
    hp = jnp.pad(h1, ((2, 2), (2, 2), (0, 0)))
    xf = jnp.concatenate([hp[:, kw:kw + 76, :] for kw in range(5)],
                         axis=-1)
    q = jnp.dot(xf.reshape(80 * 76, 160), w2_ref[...],
                preferred_element_type=_F32)
    q = q.reshape(80, 76, 320)
    a2 = (q[0:76, :, 0:64] + q[1:77, :, 64:128] + q[2:78, :, 128:192]
          + q[3:79, :, 192:256] + q[4:80, :, 256:320])
    h2 = _pool2(a2)
    h2 = jnp.maximum(h2 + b2_ref[...], 0.0).astype(_BF)
    o_ref[0] = h2.reshape(38, 38 * 64)


def _conv_block(x, wk, b, Ho, B):
    KH = wk.shape[0]
    Cout = wk.shape[2]
    xf = jnp.concatenate([x[:, :, kw:kw + Ho, :] for kw in range(KH)],
                         axis=-1)
    acc = jnp.dot(xf[:, 0:Ho].reshape(B * Ho * Ho, -1), wk[0],
                  preferred_element_type=_F32)
    for kh in range(1, KH):
        acc = acc + jnp.dot(xf[:, kh:kh + Ho].reshape(B * Ho * Ho, -1), wk[kh],
                            preferred_element_type=_F32)
    h = _pool2(acc.reshape(B, Ho, Ho, Cout))
    return jnp.maximum(h + b, 0.0).astype(_BF)


def _stage345_kernel(x_ref, w3_ref, b3_ref, w4_ref, b4_ref, w5_ref, b5_ref,
                     o_ref):
    x = x_ref[...].reshape(8, 38, 38, 64)
    h3 = _conv_block(x, w3_ref[...], b3_ref[...], 36, 8)
    h4 = _conv_block(h3, w4_ref[...], b4_ref[...], 16, 8)
    h5 = _conv_block(h4, w5_ref[...], b5_ref[...], 6, 8)
    o_ref[...] = jnp.transpose(h5.reshape(8, 9, 428),
                               (0, 2, 1)).reshape(8, 3852)


def _head_kernel(x_ref, w1_ref, b1_ref, w2_ref, b2_ref, w3_ref, b3_ref, o_ref):
    h = jnp.dot(x_ref[...], w1_ref[...], preferred_element_type=_F32)
    h = jnp.maximum(h + b1_ref[...], 0.0).astype(_BF)
    h = jnp.dot(h, w2_ref[...], preferred_element_type=_F32)
    h = jnp.maximum(h + b2_ref[...], 0.0).astype(_BF)
    h = jnp.dot(h, w3_ref[...], preferred_element_type=_F32)
    o_ref[...] = (h + b3_ref[...]).astype(o_ref.dtype)


def kernel(x_nchw, w1, b1, w2, b2, w3, b3, w4, b4, w5, b5,
           wfc1, bfc1, wfc2, bfc2, wout, bout):
    N = x_nchw.shape[0]
    xb = x_nchw.astype(_BF).reshape(N, 3, 76, 304)
    s2d = jnp.asarray(_S2D_NP).astype(_BF)

    w1p = _pack_w1(w1.astype(_BF))
    w2p = jnp.transpose(w2, (1, 2, 0, 3)).reshape(160, 5 * 64).astype(_BF)
    wk3 = w3.reshape(3, 3 * 64, 128).astype(_BF)
    wk4 = w4.reshape(3, 3 * 128, 256).astype(_BF)
    wk5 = w5.reshape(3, 3 * 256, 428).astype(_BF)

    h2 = pl.pallas_call(
        _stage12_kernel,
        out_shape=jax.ShapeDtypeStruct((N, 38, 38 * 64), _BF),
        grid=(N,),
        in_specs=[
            pl.BlockSpec((1, 3, 76, 304), lambda n: (n, 0, 0, 0)),
            pl.BlockSpec((912, 2736), lambda n: (0, 0)),
            pl.BlockSpec((108, 128), lambda n: (0, 0)),
            pl.BlockSpec((1, 32), lambda n: (0, 0)),
            pl.BlockSpec((160, 320), lambda n: (0, 0)),
            pl.BlockSpec((1, 64), lambda n: (0, 0)),
        ],
        out_specs=pl.BlockSpec((1, 38, 2432), lambda n: (n, 0, 0)),
        compiler_params=pltpu.CompilerParams(
            dimension_semantics=("parallel",),
            vmem_limit_bytes=100 * 1024 * 1024,
        ),
    )(xb, s2d, w1p, b1.reshape(1, 32), w2p, b2.reshape(1, 64))

    B = 8
    flat = pl.pallas_call(
        _stage345_kernel,
        out_shape=jax.ShapeDtypeStruct((N, 3852), _BF),
        grid=(N // B,),
        in_specs=[
            pl.BlockSpec((B, 38, 2432), lambda c: (c, 0, 0)),
            pl.BlockSpec((3, 192, 128), lambda c: (0, 0, 0)),
            pl.BlockSpec((1, 128), lambda c: (0, 0)),
            pl.BlockSpec((3, 384, 256), lambda c: (0, 0, 0)),
            pl.BlockSpec((1, 256), lambda c: (0, 0)),
            pl.BlockSpec((3, 768, 428), lambda c: (0, 0, 0)),
            pl.BlockSpec((1, 428), lambda c: (0, 0)),
        ],
        out_specs=pl.BlockSpec((B, 3852), lambda c: (c, 0)),
        compiler_params=pltpu.CompilerParams(
            dimension_semantics=("parallel",),
            vmem_limit_bytes=100 * 1024 * 1024,
        ),
    )(h2, wk3, b3.reshape(1, 128), wk4, b4.reshape(1, 256),
      wk5, b5.reshape(1, 428))

    n_out = wout.shape[1]
    out = pl.pallas_call(
        _head_kernel,
        out_shape=jax.ShapeDtypeStruct((N, n_out), _F32),
        grid=(1,),
        in_specs=[
            pl.BlockSpec((N, 3852), lambda i: (0, 0)),
            pl.BlockSpec((3852, 1024), lambda i: (0, 0)),
            pl.BlockSpec((1, 1024), lambda i: (0, 0)),
            pl.BlockSpec((1024, 128), lambda i: (0, 0)),
            pl.BlockSpec((1, 128), lambda i: (0, 0)),
            pl.BlockSpec((128, n_out), lambda i: (0, 0)),
            pl.BlockSpec((1, n_out), lambda i: (0, 0)),
        ],
        out_specs=pl.BlockSpec((N, n_out), lambda i: (0, 0)),
        compiler_params=pltpu.CompilerParams(
            dimension_semantics=("arbitrary",),
            vmem_limit_bytes=100 * 1024 * 1024,
        ),
    )(flat, wfc1.astype(_BF), bfc1.reshape(1, -1),
      wfc2.astype(_BF), bfc2.reshape(1, -1), wout.astype(_BF),
      bout.reshape(1, -1))
    return out
